# baseline (speedup 1.0000x reference)
import functools

import jax
import jax.numpy as jnp
from jax import lax
from jax.experimental import pallas as pl
from jax.experimental.pallas import tpu as pltpu
from jax.experimental.pallas import tpu_sc as plsc

_NUM_EMBED = 1024
_EMBED_DIM = 64
_BETA = 0.25

_N_ROWS = 64 * 576
_BLK = 512
_N_BLOCKS = _N_ROWS // _BLK

_SC_CORES = 2
_SC_SUBCORES = 16
_NW = _SC_CORES * _SC_SUBCORES
_BPW = _N_ROWS // _NW
_IDX_CHUNK = 128
_CHUNKS = _BPW // _IDX_CHUNK


def _tc_body(z_ref, emb_ref, idx_ref, loss_ref, ppl_ref, loss_acc, cnt_acc):
    step = pl.program_id(0)

    z = z_ref[...]
    e = emb_ref[...]

    esq = lax.dot_general(
        jnp.ones((1, _EMBED_DIM), jnp.float32), e * e,
        (((1,), (1,)), ((), ())),
        preferred_element_type=jnp.float32,
        precision=lax.Precision.HIGHEST,
    )
    prod = lax.dot_general(
        z.astype(jnp.bfloat16), e.astype(jnp.bfloat16),
        (((1,), (1,)), ((), ())),
        preferred_element_type=jnp.float32,
    )
    d = prod * -2.0 + esq

    dmin = jnp.min(d, axis=1, keepdims=True)
    hits = jnp.where(d == dmin, jnp.float32(1), jnp.float32(0))

    iota_col = lax.broadcasted_iota(
        jnp.int32, (_NUM_EMBED, 1), 0).astype(jnp.float32)
    idxf = lax.dot_general(
        hits, iota_col, (((1,), (0,)), ((), ())),
        preferred_element_type=jnp.float32,
        precision=lax.Precision.HIGHEST,
    )
    idx_ref[...] = idxf.astype(jnp.int32)
    cnt = lax.dot_general(
        jnp.ones((1, z.shape[0]), jnp.float32), hits,
        (((1,), (0,)), ((), ())),
        preferred_element_type=jnp.float32,
        precision=lax.Precision.HIGHEST,
    )

    @pl.when(step == 0)
    def _init():
        loss_acc[...] = jnp.zeros_like(loss_acc)
        cnt_acc[...] = jnp.zeros_like(cnt_acc)

    loss_acc[...] += (jnp.sum(dmin, axis=(0, 1), keepdims=True)
                      + jnp.sum(z * z).reshape(1, 1))
    cnt_acc[...] += cnt

    @pl.when(step == pl.num_programs(0) - 1)
    def _finalize():
        loss_ref[...] = _BETA * loss_acc[...] / (_N_ROWS * _EMBED_DIM)
        p = cnt_acc[...] / _N_ROWS
        ent = jnp.sum(p * jnp.log(p + 1e-10), axis=(0, 1), keepdims=True)
        ppl_ref[...] = jnp.exp(-ent)


_tc_call = pl.pallas_call(
    _tc_body,
    grid=(_N_BLOCKS,),
    in_specs=[
        pl.BlockSpec((_BLK, _EMBED_DIM), lambda i: (i, 0)),
        pl.BlockSpec((_NUM_EMBED, _EMBED_DIM), lambda i: (0, 0)),
    ],
    out_specs=[
        pl.BlockSpec((_BLK, 1), lambda i: (i, 0)),
        pl.BlockSpec((1, 1), lambda i: (0, 0)),
        pl.BlockSpec((1, 1), lambda i: (0, 0)),
    ],
    out_shape=[
        jax.ShapeDtypeStruct((_N_ROWS, 1), jnp.int32),
        jax.ShapeDtypeStruct((1, 1), jnp.float32),
        jax.ShapeDtypeStruct((1, 1), jnp.float32),
    ],
    scratch_shapes=[
        pltpu.VMEM((1, 1), jnp.float32),
        pltpu.VMEM((1, _NUM_EMBED), jnp.float32),
    ],
)


@functools.lru_cache(maxsize=1)
def _make_sc_gather():
    @functools.partial(
        pl.kernel,
        out_type=jax.ShapeDtypeStruct((_N_ROWS, _EMBED_DIM), jnp.float32),
        mesh=plsc.VectorSubcoreMesh(
            core_axis_name="c", subcore_axis_name="s",
            num_cores=_SC_CORES, num_subcores=_SC_SUBCORES),
        scratch_types=[
            pltpu.VMEM((_CHUNKS, _IDX_CHUNK), jnp.int32),
            pltpu.VMEM((_BPW, _EMBED_DIM), jnp.float32),
            pltpu.SemaphoreType.DMA,
        ],
        compiler_params=pltpu.CompilerParams(use_tc_tiling_on_sc=False),
    )
    def _sc_gather(idx_hbm, table_hbm, out_hbm, idx_v, rows_v, sem):
        wid = lax.axis_index("s") * _SC_CORES + lax.axis_index("c")
        pltpu.sync_copy(idx_hbm.at[wid], idx_v)
        copies = []
        for j in range(_CHUNKS):
            copies.append(pltpu.async_copy(
                table_hbm.at[idx_v.at[j]],
                rows_v.at[pl.ds(j * _IDX_CHUNK, _IDX_CHUNK)],
                sem))
        for c in copies:
            c.wait()
        pltpu.sync_copy(rows_v, out_hbm.at[pl.ds(wid * _BPW, _BPW)])

    return _sc_gather


def kernel(z, embedding):
    z_flat = z.reshape(_N_ROWS, _EMBED_DIM)
    idx2d, loss, ppl = _tc_call(z_flat, embedding)
    encoding_indices = idx2d.reshape(_N_ROWS)
    table = embedding.astype(jnp.bfloat16).astype(jnp.float32)
    z_q = _make_sc_gather()(
        encoding_indices.reshape(_NW, _CHUNKS, _IDX_CHUNK), table)
    return (z_q.reshape(z.shape), loss.reshape(()), ppl.reshape(()),
            encoding_indices)


# speedup vs baseline: 1.7065x; 1.7065x over previous
import functools

import jax
import jax.numpy as jnp
from jax import lax
from jax.experimental import pallas as pl
from jax.experimental.pallas import tpu as pltpu
from jax.experimental.pallas import tpu_sc as plsc

_NUM_EMBED = 1024
_EMBED_DIM = 64
_BETA = 0.25

_N_ROWS = 64 * 576
_BLK = 512
_N_BLOCKS = _N_ROWS // _BLK

_SC_CORES = 2
_SC_SUBCORES = 16
_NW = _SC_CORES * _SC_SUBCORES
_BPW = _N_ROWS // _NW
_IDX_CHUNK = 128
_CHUNKS = _BPW // _IDX_CHUNK


def _tc_body(z_ref, emb_ref, idx_ref, loss_ref, ppl_ref, loss_acc, cnt_acc):
    step = pl.program_id(0)

    z = z_ref[...]
    e = emb_ref[...]

    esq = lax.dot_general(
        jnp.ones((1, _EMBED_DIM), jnp.float32), e * e,
        (((1,), (1,)), ((), ())),
        preferred_element_type=jnp.float32,
        precision=lax.Precision.HIGHEST,
    )
    prod = lax.dot_general(
        z.astype(jnp.bfloat16), e.astype(jnp.bfloat16),
        (((1,), (1,)), ((), ())),
        preferred_element_type=jnp.float32,
    )
    d = prod * -2.0 + esq

    dmin = jnp.min(d, axis=1, keepdims=True)
    hits = jnp.where(d == dmin, jnp.float32(1),
                     jnp.float32(0)).astype(jnp.bfloat16)

    iota_f = lax.broadcasted_iota(
        jnp.int32, (_NUM_EMBED, 1), 0).astype(jnp.float32)
    iota_hi = iota_f.astype(jnp.bfloat16)
    iota_lo = (iota_f - iota_hi.astype(jnp.float32)).astype(jnp.bfloat16)
    iota_w = jnp.concatenate([iota_hi, iota_lo], axis=1)
    idxf2 = lax.dot_general(
        hits, iota_w, (((1,), (0,)), ((), ())),
        preferred_element_type=jnp.float32,
    )
    idxf = idxf2[:, 0:1] + idxf2[:, 1:2]
    idx_ref[...] = idxf.astype(jnp.int32)
    cnt = lax.dot_general(
        jnp.ones((1, z.shape[0]), jnp.bfloat16), hits,
        (((1,), (0,)), ((), ())),
        preferred_element_type=jnp.float32,
    )

    @pl.when(step == 0)
    def _init():
        loss_acc[...] = jnp.zeros_like(loss_acc)
        cnt_acc[...] = jnp.zeros_like(cnt_acc)

    loss_acc[...] += (jnp.sum(dmin, axis=(0, 1), keepdims=True)
                      + jnp.sum(z * z).reshape(1, 1))
    cnt_acc[...] += cnt

    @pl.when(step == pl.num_programs(0) - 1)
    def _finalize():
        loss_ref[...] = _BETA * loss_acc[...] / (_N_ROWS * _EMBED_DIM)
        p = cnt_acc[...] / _N_ROWS
        ent = jnp.sum(p * jnp.log(p + 1e-10), axis=(0, 1), keepdims=True)
        ppl_ref[...] = jnp.exp(-ent)


_tc_call = pl.pallas_call(
    _tc_body,
    grid=(_N_BLOCKS,),
    in_specs=[
        pl.BlockSpec((_BLK, _EMBED_DIM), lambda i: (i, 0)),
        pl.BlockSpec((_NUM_EMBED, _EMBED_DIM), lambda i: (0, 0)),
    ],
    out_specs=[
        pl.BlockSpec((_BLK, 1), lambda i: (i, 0)),
        pl.BlockSpec((1, 1), lambda i: (0, 0)),
        pl.BlockSpec((1, 1), lambda i: (0, 0)),
    ],
    out_shape=[
        jax.ShapeDtypeStruct((_N_ROWS, 1), jnp.int32),
        jax.ShapeDtypeStruct((1, 1), jnp.float32),
        jax.ShapeDtypeStruct((1, 1), jnp.float32),
    ],
    scratch_shapes=[
        pltpu.VMEM((1, 1), jnp.float32),
        pltpu.VMEM((1, _NUM_EMBED), jnp.float32),
    ],
)


@functools.lru_cache(maxsize=1)
def _make_sc_gather():
    @functools.partial(
        pl.kernel,
        out_type=jax.ShapeDtypeStruct((_N_ROWS, _EMBED_DIM), jnp.float32),
        mesh=plsc.VectorSubcoreMesh(
            core_axis_name="c", subcore_axis_name="s",
            num_cores=_SC_CORES, num_subcores=_SC_SUBCORES),
        scratch_types=[
            pltpu.VMEM((_CHUNKS, _IDX_CHUNK), jnp.int32),
            pltpu.VMEM((_BPW, _EMBED_DIM), jnp.float32),
            pltpu.SemaphoreType.DMA,
        ],
        compiler_params=pltpu.CompilerParams(use_tc_tiling_on_sc=False),
    )
    def _sc_gather(idx_hbm, table_hbm, out_hbm, idx_v, rows_v, sem):
        wid = lax.axis_index("s") * _SC_CORES + lax.axis_index("c")
        pltpu.sync_copy(idx_hbm.at[wid], idx_v)
        copies = []
        for j in range(_CHUNKS):
            copies.append(pltpu.async_copy(
                table_hbm.at[idx_v.at[j]],
                rows_v.at[pl.ds(j * _IDX_CHUNK, _IDX_CHUNK)],
                sem))
        for c in copies:
            c.wait()
        pltpu.sync_copy(rows_v, out_hbm.at[pl.ds(wid * _BPW, _BPW)])

    return _sc_gather


def kernel(z, embedding):
    z_flat = z.reshape(_N_ROWS, _EMBED_DIM)
    idx2d, loss, ppl = _tc_call(z_flat, embedding)
    encoding_indices = idx2d.reshape(_N_ROWS)
    table = embedding.astype(jnp.bfloat16).astype(jnp.float32)
    z_q = _make_sc_gather()(
        encoding_indices.reshape(_NW, _CHUNKS, _IDX_CHUNK), table)
    return (z_q.reshape(z.shape), loss.reshape(()), ppl.reshape(()),
            encoding_indices)


# speedup vs baseline: 2.1098x; 1.2363x over previous
import functools

import jax
import jax.numpy as jnp
from jax import lax
from jax.experimental import pallas as pl
from jax.experimental.pallas import tpu as pltpu
from jax.experimental.pallas import tpu_sc as plsc

_NUM_EMBED = 1024
_EMBED_DIM = 64
_BETA = 0.25

_N_ROWS = 64 * 576
_BLK = 1024
_N_BLOCKS = _N_ROWS // _BLK

_SC_CORES = 2
_SC_SUBCORES = 16
_NW = _SC_CORES * _SC_SUBCORES
_BPW = _N_ROWS // _NW
_IDX_CHUNK = 128
_CHUNKS = _BPW // _IDX_CHUNK


def _tc_body(z_ref, emb_ref, idx_ref, loss_ref, ppl_ref, tab_ref,
             loss_acc, cnt_acc):
    step = pl.program_id(0)

    z = z_ref[...]
    e = emb_ref[...]
    e16 = e.astype(jnp.bfloat16)

    @pl.when(step == 0)
    def _emit_table():
        tab_ref[...] = e16.astype(jnp.float32)

    esq = lax.dot_general(
        jnp.ones((1, _EMBED_DIM), jnp.float32), e * e,
        (((1,), (1,)), ((), ())),
        preferred_element_type=jnp.float32,
        precision=lax.Precision.HIGHEST,
    )
    prod = lax.dot_general(
        z.astype(jnp.bfloat16), e16,
        (((1,), (1,)), ((), ())),
        preferred_element_type=jnp.float32,
    )
    d = prod * -2.0 + esq

    dmin = jnp.min(d, axis=1, keepdims=True)
    hits = jnp.where(d == dmin, jnp.float32(1),
                     jnp.float32(0)).astype(jnp.bfloat16)

    iota_f = lax.broadcasted_iota(
        jnp.int32, (_NUM_EMBED, 1), 0).astype(jnp.float32)
    iota_hi = iota_f.astype(jnp.bfloat16)
    iota_lo = (iota_f - iota_hi.astype(jnp.float32)).astype(jnp.bfloat16)
    iota_w = jnp.concatenate([iota_hi, iota_lo], axis=1)
    idxf2 = lax.dot_general(
        hits, iota_w, (((1,), (0,)), ((), ())),
        preferred_element_type=jnp.float32,
    )
    idxf = idxf2[:, 0:1] + idxf2[:, 1:2]
    idx_ref[...] = idxf.astype(jnp.int32).reshape(1, _BLK // 128, 128)
    cnt = lax.dot_general(
        jnp.ones((1, z.shape[0]), jnp.bfloat16), hits,
        (((1,), (0,)), ((), ())),
        preferred_element_type=jnp.float32,
    )

    @pl.when(step == 0)
    def _init():
        loss_acc[...] = jnp.zeros_like(loss_acc)
        cnt_acc[...] = jnp.zeros_like(cnt_acc)

    loss_acc[...] += (jnp.sum(dmin, axis=(0, 1), keepdims=True)
                      + jnp.sum(z * z).reshape(1, 1))
    cnt_acc[...] += cnt

    @pl.when(step == pl.num_programs(0) - 1)
    def _finalize():
        loss_ref[...] = _BETA * loss_acc[...] / (_N_ROWS * _EMBED_DIM)
        p = cnt_acc[...] / _N_ROWS
        ent = jnp.sum(p * jnp.log(p + 1e-10), axis=(0, 1), keepdims=True)
        ppl_ref[...] = jnp.exp(-ent)


_tc_call = pl.pallas_call(
    _tc_body,
    grid=(_N_BLOCKS,),
    in_specs=[
        pl.BlockSpec((_BLK, _EMBED_DIM), lambda i: (i, 0)),
        pl.BlockSpec((_NUM_EMBED, _EMBED_DIM), lambda i: (0, 0)),
    ],
    out_specs=[
        pl.BlockSpec((1, _BLK // 128, 128), lambda i: (i, 0, 0)),
        pl.BlockSpec((1, 1), lambda i: (0, 0)),
        pl.BlockSpec((1, 1), lambda i: (0, 0)),
        pl.BlockSpec((_NUM_EMBED, _EMBED_DIM), lambda i: (0, 0)),
    ],
    out_shape=[
        jax.ShapeDtypeStruct((_N_BLOCKS, _BLK // 128, 128), jnp.int32),
        jax.ShapeDtypeStruct((1, 1), jnp.float32),
        jax.ShapeDtypeStruct((1, 1), jnp.float32),
        jax.ShapeDtypeStruct((_NUM_EMBED, _EMBED_DIM), jnp.float32),
    ],
    scratch_shapes=[
        pltpu.VMEM((1, 1), jnp.float32),
        pltpu.VMEM((1, _NUM_EMBED), jnp.float32),
    ],
)


@functools.lru_cache(maxsize=1)
def _make_sc_gather():
    @functools.partial(
        pl.kernel,
        out_type=jax.ShapeDtypeStruct((_N_ROWS, _EMBED_DIM), jnp.float32),
        mesh=plsc.VectorSubcoreMesh(
            core_axis_name="c", subcore_axis_name="s",
            num_cores=_SC_CORES, num_subcores=_SC_SUBCORES),
        scratch_types=[
            pltpu.VMEM((_CHUNKS, _IDX_CHUNK), jnp.int32),
            pltpu.VMEM((_BPW, _EMBED_DIM), jnp.float32),
            pltpu.SemaphoreType.DMA,
        ],
        compiler_params=pltpu.CompilerParams(use_tc_tiling_on_sc=False),
    )
    def _sc_gather(idx_hbm, table_hbm, out_hbm, idx_v, rows_v, sem):
        wid = lax.axis_index("s") * _SC_CORES + lax.axis_index("c")
        pltpu.sync_copy(idx_hbm.at[wid], idx_v)
        copies = []
        for j in range(_CHUNKS):
            copies.append(pltpu.async_copy(
                table_hbm.at[idx_v.at[j]],
                rows_v.at[pl.ds(j * _IDX_CHUNK, _IDX_CHUNK)],
                sem))
        for c in copies:
            c.wait()
        pltpu.sync_copy(rows_v, out_hbm.at[pl.ds(wid * _BPW, _BPW)])

    return _sc_gather


def kernel(z, embedding):
    z_flat = z.reshape(_N_ROWS, _EMBED_DIM)
    idx3d, loss, ppl, table = _tc_call(z_flat, embedding)
    encoding_indices = idx3d.reshape(_N_ROWS)
    z_q = _make_sc_gather()(
        idx3d.reshape(_NW, _CHUNKS, _IDX_CHUNK), table)
    return (z_q.reshape(z.shape), loss.reshape(()), ppl.reshape(()),
            encoding_indices)


# speedup vs baseline: 2.2359x; 1.0597x over previous
import functools

import jax
import jax.numpy as jnp
from jax import lax
from jax.experimental import pallas as pl
from jax.experimental.pallas import tpu as pltpu
from jax.experimental.pallas import tpu_sc as plsc

_NUM_EMBED = 1024
_EMBED_DIM = 64
_BETA = 0.25

_N_ROWS = 64 * 576
_BLK = 1024
_N_BLOCKS = _N_ROWS // _BLK

_SC_CORES = 2
_SC_SUBCORES = 16
_NW = _SC_CORES * _SC_SUBCORES
_BPW = _N_ROWS // _NW
_IDX_CHUNK = 128
_CHUNKS = _BPW // _IDX_CHUNK


def _tc_body(z_ref, emb_ref, idx_ref, loss_ref, ppl_ref, tab_ref,
             loss_acc, cnt_acc):
    step = pl.program_id(0)

    z = z_ref[...]
    e = emb_ref[...]
    e16 = e.astype(jnp.bfloat16)

    @pl.when(step == 0)
    def _emit_table():
        tab_ref[...] = e16.astype(jnp.float32)

    esq = jnp.sum(e * e, axis=1, keepdims=True)
    esq_hi = esq.astype(jnp.bfloat16)
    r1 = esq - esq_hi.astype(jnp.float32)
    esq_mid = r1.astype(jnp.bfloat16)
    esq_lo = (r1 - esq_mid.astype(jnp.float32)).astype(jnp.bfloat16)
    w_aug = jnp.concatenate(
        [e16 * jnp.bfloat16(-2), esq_hi, esq_mid, esq_lo], axis=1)
    z_aug = jnp.concatenate(
        [z.astype(jnp.bfloat16), jnp.ones((z.shape[0], 3), jnp.bfloat16)],
        axis=1)
    d = lax.dot_general(
        z_aug, w_aug, (((1,), (1,)), ((), ())),
        preferred_element_type=jnp.float32,
    )

    dmin = jnp.min(d, axis=1, keepdims=True)
    hits = jnp.where(d == dmin, jnp.float32(1),
                     jnp.float32(0)).astype(jnp.bfloat16)

    iota_f = lax.broadcasted_iota(
        jnp.int32, (_NUM_EMBED, 1), 0).astype(jnp.float32)
    iota_hi = iota_f.astype(jnp.bfloat16)
    iota_lo = (iota_f - iota_hi.astype(jnp.float32)).astype(jnp.bfloat16)
    iota_w = jnp.concatenate([iota_hi, iota_lo], axis=1)
    idxf2 = lax.dot_general(
        hits, iota_w, (((1,), (0,)), ((), ())),
        preferred_element_type=jnp.float32,
    )
    idxf = idxf2[:, 0:1] + idxf2[:, 1:2]
    idx_ref[...] = idxf.astype(jnp.int32).reshape(1, _BLK // 128, 128)
    cnt = lax.dot_general(
        jnp.ones((1, z.shape[0]), jnp.bfloat16), hits,
        (((1,), (0,)), ((), ())),
        preferred_element_type=jnp.float32,
    )

    @pl.when(step == 0)
    def _init():
        loss_acc[...] = jnp.zeros_like(loss_acc)
        cnt_acc[...] = jnp.zeros_like(cnt_acc)

    loss_acc[...] += (jnp.sum(dmin, axis=(0, 1), keepdims=True)
                      + jnp.sum(z * z).reshape(1, 1))
    cnt_acc[...] += cnt

    @pl.when(step == pl.num_programs(0) - 1)
    def _finalize():
        loss_ref[...] = _BETA * loss_acc[...] / (_N_ROWS * _EMBED_DIM)
        p = cnt_acc[...] / _N_ROWS
        ent = jnp.sum(p * jnp.log(p + 1e-10), axis=(0, 1), keepdims=True)
        ppl_ref[...] = jnp.exp(-ent)


_tc_call = pl.pallas_call(
    _tc_body,
    grid=(_N_BLOCKS,),
    in_specs=[
        pl.BlockSpec((_BLK, _EMBED_DIM), lambda i: (i, 0)),
        pl.BlockSpec((_NUM_EMBED, _EMBED_DIM), lambda i: (0, 0)),
    ],
    out_specs=[
        pl.BlockSpec((1, _BLK // 128, 128), lambda i: (i, 0, 0)),
        pl.BlockSpec((1, 1), lambda i: (0, 0)),
        pl.BlockSpec((1, 1), lambda i: (0, 0)),
        pl.BlockSpec((_NUM_EMBED, _EMBED_DIM), lambda i: (0, 0)),
    ],
    out_shape=[
        jax.ShapeDtypeStruct((_N_BLOCKS, _BLK // 128, 128), jnp.int32),
        jax.ShapeDtypeStruct((1, 1), jnp.float32),
        jax.ShapeDtypeStruct((1, 1), jnp.float32),
        jax.ShapeDtypeStruct((_NUM_EMBED, _EMBED_DIM), jnp.float32),
    ],
    scratch_shapes=[
        pltpu.VMEM((1, 1), jnp.float32),
        pltpu.VMEM((1, _NUM_EMBED), jnp.float32),
    ],
)


@functools.lru_cache(maxsize=1)
def _make_sc_gather():
    @functools.partial(
        pl.kernel,
        out_type=jax.ShapeDtypeStruct((_N_ROWS, _EMBED_DIM), jnp.float32),
        mesh=plsc.VectorSubcoreMesh(
            core_axis_name="c", subcore_axis_name="s",
            num_cores=_SC_CORES, num_subcores=_SC_SUBCORES),
        scratch_types=[
            pltpu.VMEM((_CHUNKS, _IDX_CHUNK), jnp.int32),
            pltpu.VMEM((_BPW, _EMBED_DIM), jnp.float32),
            pltpu.SemaphoreType.DMA,
        ],
        compiler_params=pltpu.CompilerParams(use_tc_tiling_on_sc=False),
    )
    def _sc_gather(idx_hbm, table_hbm, out_hbm, idx_v, rows_v, sem):
        wid = lax.axis_index("s") * _SC_CORES + lax.axis_index("c")
        pltpu.sync_copy(idx_hbm.at[wid], idx_v)
        copies = []
        for j in range(_CHUNKS):
            copies.append(pltpu.async_copy(
                table_hbm.at[idx_v.at[j]],
                rows_v.at[pl.ds(j * _IDX_CHUNK, _IDX_CHUNK)],
                sem))
        for c in copies:
            c.wait()
        pltpu.sync_copy(rows_v, out_hbm.at[pl.ds(wid * _BPW, _BPW)])

    return _sc_gather


def kernel(z, embedding):
    z_flat = z.reshape(_N_ROWS, _EMBED_DIM)
    idx3d, loss, ppl, table = _tc_call(z_flat, embedding)
    encoding_indices = idx3d.reshape(_N_ROWS)
    z_q = _make_sc_gather()(
        idx3d.reshape(_NW, _CHUNKS, _IDX_CHUNK), table)
    return (z_q.reshape(z.shape), loss.reshape(()), ppl.reshape(()),
            encoding_indices)


# speedup vs baseline: 2.2380x; 1.0009x over previous
import functools

import jax
import jax.numpy as jnp
from jax import lax
from jax.experimental import pallas as pl
from jax.experimental.pallas import tpu as pltpu
from jax.experimental.pallas import tpu_sc as plsc

_NUM_EMBED = 1024
_EMBED_DIM = 64
_BETA = 0.25

_N_ROWS = 64 * 576
_BLK = 1024
_N_BLOCKS = _N_ROWS // _BLK

_SC_CORES = 2
_SC_SUBCORES = 16
_NW = _SC_CORES * _SC_SUBCORES
_BPW = _N_ROWS // _NW
_IDX_CHUNK = 128
_CHUNKS = _BPW // _IDX_CHUNK


def _tc_body(z_ref, emb_ref, idx_ref, loss_ref, ppl_ref, tab_ref,
             loss_acc, cnt_acc):
    step = pl.program_id(0)

    z = z_ref[...]
    e = emb_ref[...]
    e16 = e.astype(jnp.bfloat16)

    @pl.when(step == 0)
    def _emit_table():
        tab_ref[...] = e16.astype(jnp.float32)

    esq = jnp.sum(e * e, axis=1, keepdims=True)
    esq_hi = esq.astype(jnp.bfloat16)
    r1 = esq - esq_hi.astype(jnp.float32)
    esq_mid = r1.astype(jnp.bfloat16)
    esq_lo = (r1 - esq_mid.astype(jnp.float32)).astype(jnp.bfloat16)
    w_aug = jnp.concatenate(
        [e16 * jnp.bfloat16(-2), esq_hi, esq_mid, esq_lo], axis=1)
    z_aug = jnp.concatenate(
        [z.astype(jnp.bfloat16), jnp.ones((z.shape[0], 3), jnp.bfloat16)],
        axis=1)
    d = lax.dot_general(
        z_aug, w_aug, (((1,), (1,)), ((), ())),
        preferred_element_type=jnp.float32,
    )

    dmin = jnp.min(d, axis=1, keepdims=True)
    hits = jnp.where(d == dmin, jnp.float32(1),
                     jnp.float32(0)).astype(jnp.bfloat16)

    iota_f = lax.broadcasted_iota(
        jnp.int32, (_NUM_EMBED, 1), 0).astype(jnp.float32)
    iota_hi = iota_f.astype(jnp.bfloat16)
    iota_lo = (iota_f - iota_hi.astype(jnp.float32)).astype(jnp.bfloat16)
    iota_w = jnp.concatenate([iota_hi, iota_lo], axis=1)
    idxf2 = lax.dot_general(
        hits, iota_w, (((1,), (0,)), ((), ())),
        preferred_element_type=jnp.float32,
    )
    idxf = idxf2[:, 0:1] + idxf2[:, 1:2]
    idx_ref[...] = idxf.astype(jnp.int32).reshape(_BLK // 128, 128)
    cnt = lax.dot_general(
        jnp.ones((1, z.shape[0]), jnp.bfloat16), hits,
        (((1,), (0,)), ((), ())),
        preferred_element_type=jnp.float32,
    )

    @pl.when(step == 0)
    def _init():
        loss_acc[...] = jnp.zeros_like(loss_acc)
        cnt_acc[...] = jnp.zeros_like(cnt_acc)

    loss_acc[...] += (jnp.sum(dmin, axis=(0, 1), keepdims=True)
                      + jnp.sum(z * z).reshape(1, 1))
    cnt_acc[...] += cnt

    @pl.when(step == pl.num_programs(0) - 1)
    def _finalize():
        loss_ref[...] = _BETA * loss_acc[...] / (_N_ROWS * _EMBED_DIM)
        p = cnt_acc[...] / _N_ROWS
        ent = jnp.sum(p * jnp.log(p + 1e-10), axis=(0, 1), keepdims=True)
        ppl_ref[...] = jnp.exp(-ent)


_tc_call = pl.pallas_call(
    _tc_body,
    grid=(_N_BLOCKS,),
    in_specs=[
        pl.BlockSpec((_BLK, _EMBED_DIM), lambda i: (i, 0)),
        pl.BlockSpec((_NUM_EMBED, _EMBED_DIM), lambda i: (0, 0)),
    ],
    out_specs=[
        pl.BlockSpec((_BLK // 128, 128), lambda i: (i, 0)),
        pl.BlockSpec((1, 1), lambda i: (0, 0)),
        pl.BlockSpec((1, 1), lambda i: (0, 0)),
        pl.BlockSpec((_NUM_EMBED, _EMBED_DIM), lambda i: (0, 0)),
    ],
    out_shape=[
        jax.ShapeDtypeStruct((_N_ROWS // 128, 128), jnp.int32),
        jax.ShapeDtypeStruct((1, 1), jnp.float32),
        jax.ShapeDtypeStruct((1, 1), jnp.float32),
        jax.ShapeDtypeStruct((_NUM_EMBED, _EMBED_DIM), jnp.float32),
    ],
    scratch_shapes=[
        pltpu.VMEM((1, 1), jnp.float32),
        pltpu.VMEM((1, _NUM_EMBED), jnp.float32),
    ],
)


@functools.lru_cache(maxsize=1)
def _make_sc_gather():
    @functools.partial(
        pl.kernel,
        out_type=jax.ShapeDtypeStruct((_N_ROWS, _EMBED_DIM), jnp.float32),
        mesh=plsc.VectorSubcoreMesh(
            core_axis_name="c", subcore_axis_name="s",
            num_cores=_SC_CORES, num_subcores=_SC_SUBCORES),
        scratch_types=[
            pltpu.VMEM((_CHUNKS, _IDX_CHUNK), jnp.int32),
            pltpu.VMEM((_BPW, _EMBED_DIM), jnp.float32),
            pltpu.SemaphoreType.DMA,
        ],
        compiler_params=pltpu.CompilerParams(use_tc_tiling_on_sc=False),
    )
    def _sc_gather(idx_hbm, table_hbm, out_hbm, idx_v, rows_v, sem):
        wid = lax.axis_index("s") * _SC_CORES + lax.axis_index("c")
        pltpu.sync_copy(idx_hbm.at[pl.ds(wid * _CHUNKS, _CHUNKS)], idx_v)
        copies = []
        for j in range(_CHUNKS):
            copies.append(pltpu.async_copy(
                table_hbm.at[idx_v.at[j]],
                rows_v.at[pl.ds(j * _IDX_CHUNK, _IDX_CHUNK)],
                sem))
        for c in copies:
            c.wait()
        pltpu.sync_copy(rows_v, out_hbm.at[pl.ds(wid * _BPW, _BPW)])

    return _sc_gather


def kernel(z, embedding):
    z_flat = z.reshape(_N_ROWS, _EMBED_DIM)
    idx2d, loss, ppl, table = _tc_call(z_flat, embedding)
    encoding_indices = idx2d.reshape(_N_ROWS)
    z_q = _make_sc_gather()(idx2d, table)
    return (z_q.reshape(z.shape), loss.reshape(()), ppl.reshape(()),
            encoding_indices)


# speedup vs baseline: 2.4187x; 1.0808x over previous
import functools

import jax
import jax.numpy as jnp
from jax import lax
from jax.experimental import pallas as pl
from jax.experimental.pallas import tpu as pltpu
from jax.experimental.pallas import tpu_sc as plsc

_NUM_EMBED = 1024
_EMBED_DIM = 64
_BETA = 0.25

_N_ROWS = 64 * 576
_BLK = 2048
_N_BLOCKS = _N_ROWS // _BLK

_SC_CORES = 2
_SC_SUBCORES = 16
_NW = _SC_CORES * _SC_SUBCORES
_BPW = _N_ROWS // _NW
_IDX_CHUNK = 128
_CHUNKS = _BPW // _IDX_CHUNK


def _tc_body(z_ref, emb_ref, idx_ref, loss_ref, ppl_ref, tab_ref,
             loss_acc, cnt_acc):
    step = pl.program_id(0)

    z = z_ref[...]
    e = emb_ref[...]
    e16 = e.astype(jnp.bfloat16)

    @pl.when(step == 0)
    def _emit_table():
        tab_ref[...] = e16.astype(jnp.float32)

    esq = jnp.sum(e * e, axis=1, keepdims=True)
    esq_hi = esq.astype(jnp.bfloat16)
    r1 = esq - esq_hi.astype(jnp.float32)
    esq_mid = r1.astype(jnp.bfloat16)
    esq_lo = (r1 - esq_mid.astype(jnp.float32)).astype(jnp.bfloat16)
    w_aug = jnp.concatenate(
        [e16 * jnp.bfloat16(-2), esq_hi, esq_mid, esq_lo], axis=1)
    z_aug = jnp.concatenate(
        [z.astype(jnp.bfloat16), jnp.ones((z.shape[0], 3), jnp.bfloat16)],
        axis=1)
    d = lax.dot_general(
        z_aug, w_aug, (((1,), (1,)), ((), ())),
        preferred_element_type=jnp.float32,
    )

    dmin = jnp.min(d, axis=1, keepdims=True)
    hits = jnp.where(d == dmin, jnp.float32(1),
                     jnp.float32(0)).astype(jnp.bfloat16)

    iota_f = lax.broadcasted_iota(
        jnp.int32, (_NUM_EMBED, 1), 0).astype(jnp.float32)
    iota_hi = iota_f.astype(jnp.bfloat16)
    iota_lo = (iota_f - iota_hi.astype(jnp.float32)).astype(jnp.bfloat16)
    iota_w = jnp.concatenate([iota_hi, iota_lo], axis=1)
    idxf2 = lax.dot_general(
        hits, iota_w, (((1,), (0,)), ((), ())),
        preferred_element_type=jnp.float32,
    )
    idxf = idxf2[:, 0:1] + idxf2[:, 1:2]
    idx_ref[...] = idxf.astype(jnp.int32).reshape(_BLK // 128, 128)
    cnt = lax.dot_general(
        jnp.ones((1, z.shape[0]), jnp.bfloat16), hits,
        (((1,), (0,)), ((), ())),
        preferred_element_type=jnp.float32,
    )

    @pl.when(step == 0)
    def _init():
        loss_acc[...] = jnp.zeros_like(loss_acc)
        cnt_acc[...] = jnp.zeros_like(cnt_acc)

    loss_acc[...] += (jnp.sum(dmin, axis=(0, 1), keepdims=True)
                      + jnp.sum(z * z).reshape(1, 1))
    cnt_acc[...] += cnt

    @pl.when(step == pl.num_programs(0) - 1)
    def _finalize():
        loss_ref[...] = _BETA * loss_acc[...] / (_N_ROWS * _EMBED_DIM)
        p = cnt_acc[...] / _N_ROWS
        ent = jnp.sum(p * jnp.log(p + 1e-10), axis=(0, 1), keepdims=True)
        ppl_ref[...] = jnp.exp(-ent)


_tc_call = pl.pallas_call(
    _tc_body,
    grid=(_N_BLOCKS,),
    in_specs=[
        pl.BlockSpec((_BLK, _EMBED_DIM), lambda i: (i, 0)),
        pl.BlockSpec((_NUM_EMBED, _EMBED_DIM), lambda i: (0, 0)),
    ],
    out_specs=[
        pl.BlockSpec((_BLK // 128, 128), lambda i: (i, 0)),
        pl.BlockSpec((1, 1), lambda i: (0, 0)),
        pl.BlockSpec((1, 1), lambda i: (0, 0)),
        pl.BlockSpec((_NUM_EMBED, _EMBED_DIM), lambda i: (0, 0)),
    ],
    out_shape=[
        jax.ShapeDtypeStruct((_N_ROWS // 128, 128), jnp.int32),
        jax.ShapeDtypeStruct((1, 1), jnp.float32),
        jax.ShapeDtypeStruct((1, 1), jnp.float32),
        jax.ShapeDtypeStruct((_NUM_EMBED, _EMBED_DIM), jnp.float32),
    ],
    scratch_shapes=[
        pltpu.VMEM((1, 1), jnp.float32),
        pltpu.VMEM((1, _NUM_EMBED), jnp.float32),
    ],
)


@functools.lru_cache(maxsize=1)
def _make_sc_gather():
    @functools.partial(
        pl.kernel,
        out_type=jax.ShapeDtypeStruct((_N_ROWS, _EMBED_DIM), jnp.float32),
        mesh=plsc.VectorSubcoreMesh(
            core_axis_name="c", subcore_axis_name="s",
            num_cores=_SC_CORES, num_subcores=_SC_SUBCORES),
        scratch_types=[
            pltpu.VMEM((_CHUNKS, _IDX_CHUNK), jnp.int32),
            pltpu.VMEM((_BPW, _EMBED_DIM), jnp.float32),
            pltpu.SemaphoreType.DMA,
        ],
        compiler_params=pltpu.CompilerParams(use_tc_tiling_on_sc=False),
    )
    def _sc_gather(idx_hbm, table_hbm, out_hbm, idx_v, rows_v, sem):
        wid = lax.axis_index("s") * _SC_CORES + lax.axis_index("c")
        pltpu.sync_copy(idx_hbm.at[pl.ds(wid * _CHUNKS, _CHUNKS)], idx_v)
        copies = []
        for j in range(_CHUNKS):
            copies.append(pltpu.async_copy(
                table_hbm.at[idx_v.at[j]],
                rows_v.at[pl.ds(j * _IDX_CHUNK, _IDX_CHUNK)],
                sem))
        for c in copies:
            c.wait()
        pltpu.sync_copy(rows_v, out_hbm.at[pl.ds(wid * _BPW, _BPW)])

    return _sc_gather


def kernel(z, embedding):
    z_flat = z.reshape(_N_ROWS, _EMBED_DIM)
    idx2d, loss, ppl, table = _tc_call(z_flat, embedding)
    encoding_indices = idx2d.reshape(_N_ROWS)
    z_q = _make_sc_gather()(idx2d, table)
    return (z_q.reshape(z.shape), loss.reshape(()), ppl.reshape(()),
            encoding_indices)
